# baseline (speedup 1.0000x reference)
import functools

import jax
import jax.numpy as jnp
from jax import lax
from jax.experimental import pallas as pl
from jax.experimental.pallas import tpu as pltpu
from jax.experimental.pallas import tpu_sc as plsc

N = 10000
E = 160000
D = 256
H = 256
C = 64
K = 3

NTILE = 16
WIN = 640
N_PAD = NTILE * WIN
CB = 128
NCHUNK = 79
EPT = NCHUNK * CB
E_PAD = NTILE * EPT
HD = D // 2
GB = CB
NB = 2
NCH2 = EPT // CB
ROWCH = WIN // GB


def _sc_propagate(feats_pad, ev4):
    mesh = plsc.VectorSubcoreMesh(core_axis_name="c", subcore_axis_name="s")

    @functools.partial(
        pl.kernel,
        mesh=mesh,
        compiler_params=pltpu.CompilerParams(needs_layout_passes=False),
        out_type=[
            jax.ShapeDtypeStruct((2 * (K + 1) * N_PAD, HD), jnp.float32),
            jax.ShapeDtypeStruct((N_PAD,), jnp.float32),
        ],
        scratch_types=[
            pltpu.VMEM((NCHUNK, CB), jnp.int32),
            pltpu.VMEM((1, CB), jnp.int32),
            pltpu.VMEM((1, CB), jnp.int32),
            pltpu.VMEM((1, CB), jnp.int32),
            pltpu.VMEM((CB, HD), jnp.float32),
            pltpu.VMEM((CB, HD), jnp.float32),
            pltpu.VMEM((WIN,), jnp.float32),
            pltpu.VMEM((WIN,), jnp.float32),
            pltpu.VMEM_SHARED((N_PAD, HD), jnp.float32),
            pltpu.SemaphoreType.DMA,
            pltpu.SemaphoreType.DMA,
        ],
    )
    def body(feats_hbm, ev_hbm, xout, deg_out,
             ev, idxg0, idxg1, idxd, rows0, rows1, normw, cnt2,
             acc, sem0, sem1):
        idxg = (idxg0, idxg1)
        rows = (rows0, rows1)
        sems = (sem0, sem1)
        c = lax.axis_index("c")
        s = lax.axis_index("s")
        win0 = s * WIN

        zeros16 = jnp.zeros((16,), jnp.float32)
        ones16 = jnp.ones((16,), jnp.float32)
        zsrc = feats_hbm.at[pl.ds(N, GB), pl.ds(c * HD, HD)]

        def zdeg(i, _):
            normw[pl.ds(i * 16, 16)] = zeros16
            cnt2[pl.ds(i * 16, 16)] = zeros16
            return 0
        lax.fori_loop(0, WIN // 16, zdeg, 0)

        for k in range(ROWCH):
            pltpu.sync_copy(zsrc, acc.at[pl.ds(win0 + k * GB, GB), :])

        for t in range(NTILE):
            pltpu.sync_copy(ev_hbm.at[c, t], ev)

            def bc(i, _):
                for q in range(CB // 16):
                    d16 = ev[i, pl.ds(q * 16, 16)] >> 16
                    li = d16 - win0
                    inb = li.astype(jnp.uint32) < jnp.uint32(WIN)
                    li = jnp.where(inb, li, 0)
                    tgt = normw if q % 2 == 0 else cnt2
                    plsc.addupdate_scatter(tgt, [li], ones16, mask=inb)
                return 0
            lax.fori_loop(0, NCHUNK, bc, 0, unroll=2)
        pltpu.sync_copy(ev_hbm.at[c, s], ev)

        def clipd(j, _):
            normw[pl.ds(j * 16, 16)] = jnp.maximum(
                normw[pl.ds(j * 16, 16)] + cnt2[pl.ds(j * 16, 16)], 1.0)
            return 0
        lax.fori_loop(0, WIN // 16, clipd, 0)

        @pl.when(c == 0)
        def _():
            pltpu.sync_copy(normw, deg_out.at[pl.ds(win0, WIN)])

        def newt(j, _):
            t = normw[pl.ds(j * 16, 16)]
            ii = plsc.bitcast(t, jnp.int32)
            ii = 0x5F3759DF - (ii >> 1)
            y = plsc.bitcast(ii, jnp.float32)
            for _ in range(3):
                y = y * (1.5 - 0.5 * t * y * y)
            normw[pl.ds(j * 16, 16)] = y
            return 0
        lax.fori_loop(0, WIN // 16, newt, 0)

        for k in range(ROWCH):
            r0 = win0 + k * GB
            pltpu.sync_copy(
                feats_hbm.at[pl.ds(r0, GB), pl.ds(c * HD, HD)], rows0)

            def srow0(r, _, k=k):
                nv = plsc.load_gather(
                    normw, [jnp.full((16,), k * GB + r, jnp.int32)])
                for q in range(HD // 16):
                    rows0[r, pl.ds(q * 16, 16)] = (
                        rows0[r, pl.ds(q * 16, 16)] * nv)
                return 0
            lax.fori_loop(0, GB, srow0, 0, unroll=4)
            pltpu.sync_copy(rows0, xout.at[pl.ds(c * N_PAD + r0, GB), :])

        plsc.subcore_barrier()

        for lev in range(1, K + 1):
            tbl = xout.at[pl.ds((2 * (lev - 1)) * N_PAD, 2 * N_PAD)]

            def unpack_g(j, b):
                for q in range(CB // 16):
                    idxg[b][0, pl.ds(q * 16, 16)] = (
                        ev[j, pl.ds(q * 16, 16)] & 0xFFFF)

            def unpack_d(j):
                for q in range(CB // 16):
                    idxd[0, pl.ds(q * 16, 16)] = (
                        ev[j, pl.ds(q * 16, 16)] >> 16)

            def issue_gather(b):
                H2 = CB // 2
                pltpu.async_copy(
                    tbl.at[idxg[b].at[0, pl.ds(0, H2)]],
                    rows[b].at[pl.ds(0, H2), :], sems[b])
                pltpu.async_copy(
                    tbl.at[idxg[b].at[0, pl.ds(H2, H2)]],
                    rows[b].at[pl.ds(H2, H2), :], sems[b])

            for b in range(2):
                unpack_g(b, b)
                issue_gather(b)

            def ch(g, _, tbl=tbl):
                for b in range(2):
                    j = 2 * g + b
                    pltpu.make_async_copy(
                        tbl.at[idxg[b].at[0]], rows[b], sems[b]).wait()
                    unpack_d(j)
                    pltpu.sync_copy(rows[b], acc.at[idxd.at[0]], add=True)
                    jn = j + 2

                    @pl.when(jn < NCH2)
                    def _(b=b, jn=jn):
                        unpack_g(jn, b)
                        issue_gather(b)
                return 0
            lax.fori_loop(0, NCH2 // 2, ch, 0)

            jt = NCH2 - 1
            pltpu.make_async_copy(
                tbl.at[idxg[0].at[0]], rows[0], sems[0]).wait()
            unpack_d(jt)
            pltpu.sync_copy(rows[0], acc.at[idxd.at[0]], add=True)

            plsc.subcore_barrier()

            for k in range(ROWCH):
                r0 = win0 + k * GB
                pltpu.sync_copy(acc.at[pl.ds(r0, GB), :], rows0)

                def srow(r, _, k=k):
                    nv = plsc.load_gather(
                        normw, [jnp.full((16,), k * GB + r, jnp.int32)])
                    dv = nv * nv
                    for q in range(HD // 16):
                        rows0[r, pl.ds(q * 16, 16)] = (
                            rows0[r, pl.ds(q * 16, 16)] * dv)
                    return 0
                lax.fori_loop(0, GB, srow, 0, unroll=4)

                pltpu.sync_copy(
                    rows0, xout.at[pl.ds((2 * lev) * N_PAD + c * N_PAD + r0,
                                         GB), :])
                if lev < K:
                    pltpu.sync_copy(zsrc, acc.at[pl.ds(r0, GB), :])

            plsc.subcore_barrier()

    return body(feats_pad, ev4)


def _tc_mlp(xout8, deg_col, W1, b1r, W2, b2r):
    BN = 1000

    def mlp_body(x_ref, deg_ref, w1_ref, b1_ref, w2_ref, b2_ref, o_ref):
        h0 = x_ref[0] + x_ref[2] + x_ref[4] + x_ref[6]
        h1 = x_ref[1] + x_ref[3] + x_ref[5] + x_ref[7]
        y = jnp.concatenate([h0, h1], axis=1)
        scale = jnp.sqrt(deg_ref[...]) * 0.25
        y = y * scale
        h = lax.dot_general(y, w1_ref[...], (((1,), (1,)), ((), ())),
                            preferred_element_type=jnp.float32)
        h = jnp.maximum(h + b1_ref[...], 0.0)
        o = lax.dot_general(h, w2_ref[...], (((1,), (1,)), ((), ())),
                            preferred_element_type=jnp.float32)
        o = o + b2_ref[...]
        m = jnp.max(o, axis=1, keepdims=True)
        sh = o - m
        o_ref[...] = sh - jnp.log(jnp.sum(jnp.exp(sh), axis=1, keepdims=True))

    return pl.pallas_call(
        mlp_body,
        grid=(N // BN,),
        in_specs=[
            pl.BlockSpec((2 * (K + 1), BN, HD), lambda i: (0, i, 0)),
            pl.BlockSpec((BN, 1), lambda i: (i, 0)),
            pl.BlockSpec((H, D), lambda i: (0, 0)),
            pl.BlockSpec((1, H), lambda i: (0, 0)),
            pl.BlockSpec((C, H), lambda i: (0, 0)),
            pl.BlockSpec((1, C), lambda i: (0, 0)),
        ],
        out_specs=pl.BlockSpec((BN, C), lambda i: (i, 0)),
        out_shape=jax.ShapeDtypeStruct((N, C), jnp.float32),
    )(xout8, deg_col, W1, b1r, W2, b2r)


def kernel(feats, edge_index, W1, b1, W2, b2):
    src = edge_index[0]
    dst = edge_index[1]
    pad = E_PAD - E
    srcp = jnp.concatenate([src, jnp.zeros((pad,), jnp.int32)])
    dstp = jnp.concatenate([dst, jnp.full((pad,), N, jnp.int32)])
    srcoff = jnp.stack([srcp, srcp + N_PAD])
    ev4 = (srcoff | (dstp << 16)).reshape(2, NTILE, NCHUNK, CB)
    feats_pad = jnp.pad(feats, ((0, N_PAD - N), (0, 0)))

    xout, deg = _sc_propagate(feats_pad, ev4)

    return _tc_mlp(
        xout.reshape(2 * (K + 1), N_PAD, HD),
        deg.reshape(N_PAD, 1),
        W1, b1.reshape(1, H), W2, b2.reshape(1, C),
    )


# speedup vs baseline: 1.0375x; 1.0375x over previous
import functools

import jax
import jax.numpy as jnp
from jax import lax
from jax.experimental import pallas as pl
from jax.experimental.pallas import tpu as pltpu
from jax.experimental.pallas import tpu_sc as plsc

N = 10000
E = 160000
D = 256
H = 256
C = 64
K = 3

NTILE = 16
WIN = 640
N_PAD = NTILE * WIN
CB = 128
NCHUNK = 79
EPT = NCHUNK * CB
E_PAD = NTILE * EPT
HD = D // 2
GB = CB
NB = 2
NCH2 = EPT // CB
ROWCH = WIN // GB


def _sc_propagate(feats_pad, ev4):
    mesh = plsc.VectorSubcoreMesh(core_axis_name="c", subcore_axis_name="s")

    @functools.partial(
        pl.kernel,
        mesh=mesh,
        compiler_params=pltpu.CompilerParams(needs_layout_passes=False),
        out_type=[
            jax.ShapeDtypeStruct((2 * (K + 1) * N_PAD, HD), jnp.float32),
            jax.ShapeDtypeStruct((N_PAD,), jnp.float32),
        ],
        scratch_types=[
            pltpu.VMEM((NCHUNK, CB), jnp.int32),
            pltpu.VMEM((1, CB), jnp.int32),
            pltpu.VMEM((1, CB), jnp.int32),
            pltpu.VMEM((1, CB), jnp.int32),
            pltpu.VMEM((CB, HD), jnp.float32),
            pltpu.VMEM((CB, HD), jnp.float32),
            pltpu.VMEM((WIN,), jnp.float32),
            pltpu.VMEM((WIN,), jnp.float32),
            pltpu.VMEM_SHARED((N_PAD, HD), jnp.float32),
            pltpu.SemaphoreType.DMA,
            pltpu.SemaphoreType.DMA,
        ],
    )
    def body(feats_hbm, ev_hbm, xout, deg_out,
             ev, idxg0, idxg1, idxd, rows0, rows1, normw, cnt2,
             acc, sem0, sem1):
        idxg = (idxg0, idxg1)
        rows = (rows0, rows1)
        sems = (sem0, sem1)
        c = lax.axis_index("c")
        s = lax.axis_index("s")
        win0 = s * WIN

        zeros16 = jnp.zeros((16,), jnp.float32)
        ones16 = jnp.ones((16,), jnp.float32)
        zsrc = feats_hbm.at[pl.ds(N, GB), pl.ds(c * HD, HD)]

        def zdeg(i, _):
            normw[pl.ds(i * 16, 16)] = zeros16
            cnt2[pl.ds(i * 16, 16)] = zeros16
            return 0
        lax.fori_loop(0, WIN // 16, zdeg, 0)

        for k in range(ROWCH):
            pltpu.sync_copy(zsrc, acc.at[pl.ds(win0 + k * GB, GB), :])

        for t in range(NTILE):
            pltpu.sync_copy(ev_hbm.at[c, t], ev)

            def bc(i, _):
                for q in range(CB // 16):
                    d16 = ev[i, pl.ds(q * 16, 16)] >> 16
                    li = d16 - win0
                    inb = li.astype(jnp.uint32) < jnp.uint32(WIN)
                    li = jnp.where(inb, li, 0)
                    tgt = normw if q % 2 == 0 else cnt2
                    plsc.addupdate_scatter(tgt, [li], ones16, mask=inb)
                return 0
            lax.fori_loop(0, NCHUNK, bc, 0, unroll=2)
        pltpu.sync_copy(ev_hbm.at[c, s], ev)

        def clipd(j, _):
            normw[pl.ds(j * 16, 16)] = jnp.maximum(
                normw[pl.ds(j * 16, 16)] + cnt2[pl.ds(j * 16, 16)], 1.0)
            return 0
        lax.fori_loop(0, WIN // 16, clipd, 0)

        @pl.when(c == 0)
        def _():
            pltpu.sync_copy(normw, deg_out.at[pl.ds(win0, WIN)])

        def newt(j, _):
            t = normw[pl.ds(j * 16, 16)]
            ii = plsc.bitcast(t, jnp.int32)
            ii = 0x5F3759DF - (ii >> 1)
            y = plsc.bitcast(ii, jnp.float32)
            for _ in range(3):
                y = y * (1.5 - 0.5 * t * y * y)
            normw[pl.ds(j * 16, 16)] = y
            return 0
        lax.fori_loop(0, WIN // 16, newt, 0)

        for k in range(ROWCH):
            r0 = win0 + k * GB
            pltpu.sync_copy(
                feats_hbm.at[pl.ds(r0, GB), pl.ds(c * HD, HD)], rows0)

            def srow0(r, _, k=k):
                nv = plsc.load_gather(
                    normw, [jnp.full((16,), k * GB + r, jnp.int32)])
                for q in range(HD // 16):
                    rows0[r, pl.ds(q * 16, 16)] = (
                        rows0[r, pl.ds(q * 16, 16)] * nv)
                return 0
            lax.fori_loop(0, GB, srow0, 0, unroll=4)
            pltpu.sync_copy(rows0, xout.at[pl.ds(c * N_PAD + r0, GB), :])

        plsc.subcore_barrier()

        for lev in range(1, K + 1):
            tbl = xout.at[pl.ds((2 * (lev - 1)) * N_PAD, 2 * N_PAD)]

            def unpack_g(j, b):
                for q in range(CB // 16):
                    idxg[b][0, pl.ds(q * 16, 16)] = (
                        ev[j, pl.ds(q * 16, 16)] & 0xFFFF)

            def unpack_d(j):
                for q in range(CB // 16):
                    idxd[0, pl.ds(q * 16, 16)] = (
                        ev[j, pl.ds(q * 16, 16)] >> 16)

            def issue_gather(b):
                H2 = CB // 2
                pltpu.async_copy(
                    tbl.at[idxg[b].at[0, pl.ds(0, H2)]],
                    rows[b].at[pl.ds(0, H2), :], sems[b])
                pltpu.async_copy(
                    tbl.at[idxg[b].at[0, pl.ds(H2, H2)]],
                    rows[b].at[pl.ds(H2, H2), :], sems[b])

            for b in range(2):
                unpack_g(b, b)
                issue_gather(b)

            def ch(g, _, tbl=tbl):
                for b in range(2):
                    j = 2 * g + b
                    pltpu.make_async_copy(
                        tbl.at[idxg[b].at[0]], rows[b], sems[b]).wait()
                    jn = j + 2

                    @pl.when(jn < NCH2)
                    def _(b=b, jn=jn):
                        unpack_g(jn, b)
                        issue_gather(b)
                    unpack_d(j)
                    pltpu.sync_copy(rows[b], acc.at[idxd.at[0]], add=True)
                return 0
            lax.fori_loop(0, NCH2 // 2, ch, 0)

            jt = NCH2 - 1
            pltpu.make_async_copy(
                tbl.at[idxg[0].at[0]], rows[0], sems[0]).wait()
            unpack_d(jt)
            pltpu.sync_copy(rows[0], acc.at[idxd.at[0]], add=True)

            plsc.subcore_barrier()

            for k in range(ROWCH):
                r0 = win0 + k * GB
                pltpu.sync_copy(acc.at[pl.ds(r0, GB), :], rows0)

                def srow(r, _, k=k):
                    nv = plsc.load_gather(
                        normw, [jnp.full((16,), k * GB + r, jnp.int32)])
                    dv = nv * nv
                    for q in range(HD // 16):
                        rows0[r, pl.ds(q * 16, 16)] = (
                            rows0[r, pl.ds(q * 16, 16)] * dv)
                    return 0
                lax.fori_loop(0, GB, srow, 0, unroll=4)

                pltpu.sync_copy(
                    rows0, xout.at[pl.ds((2 * lev) * N_PAD + c * N_PAD + r0,
                                         GB), :])
                if lev < K:
                    pltpu.sync_copy(zsrc, acc.at[pl.ds(r0, GB), :])

            plsc.subcore_barrier()

    return body(feats_pad, ev4)


def _tc_mlp(xout8, deg_col, W1, b1r, W2, b2r):
    BN = 1000

    def mlp_body(x_ref, deg_ref, w1_ref, b1_ref, w2_ref, b2_ref, o_ref):
        h0 = x_ref[0] + x_ref[2] + x_ref[4] + x_ref[6]
        h1 = x_ref[1] + x_ref[3] + x_ref[5] + x_ref[7]
        y = jnp.concatenate([h0, h1], axis=1)
        scale = jnp.sqrt(deg_ref[...]) * 0.25
        y = y * scale
        h = lax.dot_general(y, w1_ref[...], (((1,), (1,)), ((), ())),
                            preferred_element_type=jnp.float32)
        h = jnp.maximum(h + b1_ref[...], 0.0)
        o = lax.dot_general(h, w2_ref[...], (((1,), (1,)), ((), ())),
                            preferred_element_type=jnp.float32)
        o = o + b2_ref[...]
        m = jnp.max(o, axis=1, keepdims=True)
        sh = o - m
        o_ref[...] = sh - jnp.log(jnp.sum(jnp.exp(sh), axis=1, keepdims=True))

    return pl.pallas_call(
        mlp_body,
        grid=(N // BN,),
        in_specs=[
            pl.BlockSpec((2 * (K + 1), BN, HD), lambda i: (0, i, 0)),
            pl.BlockSpec((BN, 1), lambda i: (i, 0)),
            pl.BlockSpec((H, D), lambda i: (0, 0)),
            pl.BlockSpec((1, H), lambda i: (0, 0)),
            pl.BlockSpec((C, H), lambda i: (0, 0)),
            pl.BlockSpec((1, C), lambda i: (0, 0)),
        ],
        out_specs=pl.BlockSpec((BN, C), lambda i: (i, 0)),
        out_shape=jax.ShapeDtypeStruct((N, C), jnp.float32),
    )(xout8, deg_col, W1, b1r, W2, b2r)


def kernel(feats, edge_index, W1, b1, W2, b2):
    src = edge_index[0]
    dst = edge_index[1]
    pad = E_PAD - E
    srcp = jnp.concatenate([src, jnp.zeros((pad,), jnp.int32)])
    dstp = jnp.concatenate([dst, jnp.full((pad,), N, jnp.int32)])
    srcoff = jnp.stack([srcp, srcp + N_PAD])
    ev4 = (srcoff | (dstp << 16)).reshape(2, NTILE, NCHUNK, CB)
    feats_pad = jnp.pad(feats, ((0, N_PAD - N), (0, 0)))

    xout, deg = _sc_propagate(feats_pad, ev4)

    return _tc_mlp(
        xout.reshape(2 * (K + 1), N_PAD, HD),
        deg.reshape(N_PAD, 1),
        W1, b1.reshape(1, H), W2, b2.reshape(1, C),
    )


# speedup vs baseline: 1.0783x; 1.0393x over previous
import functools

import jax
import jax.numpy as jnp
from jax import lax
from jax.experimental import pallas as pl
from jax.experimental.pallas import tpu as pltpu
from jax.experimental.pallas import tpu_sc as plsc

N = 10000
E = 160000
D = 256
H = 256
C = 64
K = 3

NTILE = 16
WIN = 640
N_PAD = NTILE * WIN
CB = 128
NCHUNK = 79
EPT = NCHUNK * CB
E_PAD = NTILE * EPT
HD = D // 2
GB = CB
NB = 2
NCH2 = EPT // CB
ROWCH = WIN // GB


def _sc_propagate(feats_pad, ev4, evf4):
    mesh = plsc.VectorSubcoreMesh(core_axis_name="c", subcore_axis_name="s")

    @functools.partial(
        pl.kernel,
        mesh=mesh,
        compiler_params=pltpu.CompilerParams(needs_layout_passes=False),
        out_type=[
            jax.ShapeDtypeStruct((2 * (K + 1) * N_PAD, HD), jnp.float32),
            jax.ShapeDtypeStruct((N_PAD,), jnp.float32),
        ],
        scratch_types=[
            pltpu.VMEM((NCHUNK, CB), jnp.int32),
            pltpu.VMEM((1, CB), jnp.int32),
            pltpu.VMEM((1, CB), jnp.int32),
            pltpu.VMEM((1, CB), jnp.int32),
            pltpu.VMEM((CB, HD), jnp.float32),
            pltpu.VMEM((CB, HD), jnp.float32),
            pltpu.VMEM((WIN,), jnp.float32),
            pltpu.VMEM((WIN,), jnp.float32),
            pltpu.VMEM_SHARED((N_PAD, HD), jnp.float32),
            pltpu.SemaphoreType.DMA,
            pltpu.SemaphoreType.DMA,
        ],
    )
    def body(feats_hbm, ev_hbm, evf_hbm, xout, deg_out,
             ev, idxg0, idxg1, idxd, rows0, rows1, normw, cnt2,
             acc, sem0, sem1):
        idxg = (idxg0, idxg1)
        rows = (rows0, rows1)
        sems = (sem0, sem1)
        c = lax.axis_index("c")
        s = lax.axis_index("s")
        win0 = s * WIN

        zeros16 = jnp.zeros((16,), jnp.float32)
        ones16 = jnp.ones((16,), jnp.float32)
        zsrc = feats_hbm.at[pl.ds(N, GB), pl.ds(c * HD, HD)]

        def zdeg(i, _):
            normw[pl.ds(i * 16, 16)] = zeros16
            cnt2[pl.ds(i * 16, 16)] = zeros16
            return 0
        lax.fori_loop(0, WIN // 16, zdeg, 0)

        for k in range(ROWCH):
            pltpu.sync_copy(zsrc, acc.at[pl.ds(win0 + k * GB, GB), :])

        pltpu.sync_copy(ev_hbm.at[c, s], ev)
        evst = (rows0.at[pl.ds(0, NCHUNK), :], rows1.at[pl.ds(0, NCHUNK), :])
        pltpu.sync_copy(evf_hbm.at[c, 0], evst[0])
        for t in range(NTILE):
            b = t % 2
            if t + 1 < NTILE:
                pltpu.async_copy(evf_hbm.at[c, t + 1], evst[1 - b],
                                 sems[1 - b])

            def bc(i, _, b=b):
                for q in range(CB // 16):
                    w = plsc.bitcast(evst[b][i, pl.ds(q * 16, 16)], jnp.int32)
                    li = (w >> 16) - win0
                    inb = li.astype(jnp.uint32) < jnp.uint32(WIN)
                    li = jnp.where(inb, li, 0)
                    tgt = normw if q % 2 == 0 else cnt2
                    plsc.addupdate_scatter(tgt, [li], ones16, mask=inb)
                return 0
            lax.fori_loop(0, NCHUNK, bc, 0, unroll=2)
            if t + 1 < NTILE:
                pltpu.make_async_copy(evf_hbm.at[c, t + 1], evst[1 - b],
                                      sems[1 - b]).wait()

        def clipd(j, _):
            normw[pl.ds(j * 16, 16)] = jnp.maximum(
                normw[pl.ds(j * 16, 16)] + cnt2[pl.ds(j * 16, 16)], 1.0)
            return 0
        lax.fori_loop(0, WIN // 16, clipd, 0)

        @pl.when(c == 0)
        def _():
            pltpu.sync_copy(normw, deg_out.at[pl.ds(win0, WIN)])

        def newt(j, _):
            t = normw[pl.ds(j * 16, 16)]
            ii = plsc.bitcast(t, jnp.int32)
            ii = 0x5F3759DF - (ii >> 1)
            y = plsc.bitcast(ii, jnp.float32)
            for _ in range(3):
                y = y * (1.5 - 0.5 * t * y * y)
            normw[pl.ds(j * 16, 16)] = y
            return 0
        lax.fori_loop(0, WIN // 16, newt, 0)

        for k in range(ROWCH):
            r0 = win0 + k * GB
            pltpu.sync_copy(
                feats_hbm.at[pl.ds(r0, GB), pl.ds(c * HD, HD)], rows0)

            def srow0(r, _, k=k):
                nv = plsc.load_gather(
                    normw, [jnp.full((16,), k * GB + r, jnp.int32)])
                for q in range(HD // 16):
                    rows0[r, pl.ds(q * 16, 16)] = (
                        rows0[r, pl.ds(q * 16, 16)] * nv)
                return 0
            lax.fori_loop(0, GB, srow0, 0, unroll=4)
            pltpu.sync_copy(rows0, xout.at[pl.ds(c * N_PAD + r0, GB), :])

        plsc.subcore_barrier()

        for lev in range(1, K + 1):
            tbl = xout.at[pl.ds((2 * (lev - 1)) * N_PAD, 2 * N_PAD)]

            def unpack_g(j, b):
                for q in range(CB // 16):
                    idxg[b][0, pl.ds(q * 16, 16)] = (
                        ev[j, pl.ds(q * 16, 16)] & 0xFFFF)

            def unpack_d(j):
                for q in range(CB // 16):
                    idxd[0, pl.ds(q * 16, 16)] = (
                        ev[j, pl.ds(q * 16, 16)] >> 16)

            def issue_gather(b):
                H2 = CB // 2
                pltpu.async_copy(
                    tbl.at[idxg[b].at[0, pl.ds(0, H2)]],
                    rows[b].at[pl.ds(0, H2), :], sems[b])
                pltpu.async_copy(
                    tbl.at[idxg[b].at[0, pl.ds(H2, H2)]],
                    rows[b].at[pl.ds(H2, H2), :], sems[b])

            for b in range(2):
                unpack_g(b, b)
                issue_gather(b)

            def ch(g, _, tbl=tbl):
                for b in range(2):
                    j = 2 * g + b
                    pltpu.make_async_copy(
                        tbl.at[idxg[b].at[0]], rows[b], sems[b]).wait()
                    jn = j + 2

                    @pl.when(jn < NCH2)
                    def _(b=b, jn=jn):
                        unpack_g(jn, b)
                        issue_gather(b)
                    unpack_d(j)
                    pltpu.sync_copy(rows[b], acc.at[idxd.at[0]], add=True)
                return 0
            lax.fori_loop(0, NCH2 // 2, ch, 0)

            jt = NCH2 - 1
            pltpu.make_async_copy(
                tbl.at[idxg[0].at[0]], rows[0], sems[0]).wait()
            unpack_d(jt)
            pltpu.sync_copy(rows[0], acc.at[idxd.at[0]], add=True)

            plsc.subcore_barrier()

            for k in range(ROWCH):
                r0 = win0 + k * GB
                pltpu.sync_copy(acc.at[pl.ds(r0, GB), :], rows0)

                def srow(r, _, k=k):
                    nv = plsc.load_gather(
                        normw, [jnp.full((16,), k * GB + r, jnp.int32)])
                    dv = nv * nv
                    for q in range(HD // 16):
                        rows0[r, pl.ds(q * 16, 16)] = (
                            rows0[r, pl.ds(q * 16, 16)] * dv)
                    return 0
                lax.fori_loop(0, GB, srow, 0, unroll=4)

                pltpu.sync_copy(
                    rows0, xout.at[pl.ds((2 * lev) * N_PAD + c * N_PAD + r0,
                                         GB), :])
                if lev < K:
                    pltpu.sync_copy(zsrc, acc.at[pl.ds(r0, GB), :])

            plsc.subcore_barrier()

    return body(feats_pad, ev4, evf4)


def _tc_mlp(xout8, deg_col, W1, b1r, W2, b2r):
    BN = 1000

    def mlp_body(x_ref, deg_ref, w1_ref, b1_ref, w2_ref, b2_ref, o_ref):
        h0 = x_ref[0] + x_ref[2] + x_ref[4] + x_ref[6]
        h1 = x_ref[1] + x_ref[3] + x_ref[5] + x_ref[7]
        y = jnp.concatenate([h0, h1], axis=1)
        scale = jnp.sqrt(deg_ref[...]) * 0.25
        y = y * scale
        h = lax.dot_general(y, w1_ref[...], (((1,), (1,)), ((), ())),
                            preferred_element_type=jnp.float32)
        h = jnp.maximum(h + b1_ref[...], 0.0)
        o = lax.dot_general(h, w2_ref[...], (((1,), (1,)), ((), ())),
                            preferred_element_type=jnp.float32)
        o = o + b2_ref[...]
        m = jnp.max(o, axis=1, keepdims=True)
        sh = o - m
        o_ref[...] = sh - jnp.log(jnp.sum(jnp.exp(sh), axis=1, keepdims=True))

    return pl.pallas_call(
        mlp_body,
        grid=(N // BN,),
        in_specs=[
            pl.BlockSpec((2 * (K + 1), BN, HD), lambda i: (0, i, 0)),
            pl.BlockSpec((BN, 1), lambda i: (i, 0)),
            pl.BlockSpec((H, D), lambda i: (0, 0)),
            pl.BlockSpec((1, H), lambda i: (0, 0)),
            pl.BlockSpec((C, H), lambda i: (0, 0)),
            pl.BlockSpec((1, C), lambda i: (0, 0)),
        ],
        out_specs=pl.BlockSpec((BN, C), lambda i: (i, 0)),
        out_shape=jax.ShapeDtypeStruct((N, C), jnp.float32),
    )(xout8, deg_col, W1, b1r, W2, b2r)


def kernel(feats, edge_index, W1, b1, W2, b2):
    src = edge_index[0]
    dst = edge_index[1]
    pad = E_PAD - E
    srcp = jnp.concatenate([src, jnp.zeros((pad,), jnp.int32)])
    dstp = jnp.concatenate([dst, jnp.full((pad,), N, jnp.int32)])
    srcoff = jnp.stack([srcp, srcp + N_PAD])
    ev4 = (srcoff | (dstp << 16)).reshape(2, NTILE, NCHUNK, CB)
    feats_pad = jnp.pad(feats, ((0, N_PAD - N), (0, 0)))

    xout, deg = _sc_propagate(
        feats_pad, ev4, lax.bitcast_convert_type(ev4, jnp.float32))

    return _tc_mlp(
        xout.reshape(2 * (K + 1), N_PAD, HD),
        deg.reshape(N_PAD, 1),
        W1, b1.reshape(1, H), W2, b2.reshape(1, C),
    )
